# baseline (speedup 1.0000x reference)
import dataclasses
import functools

import jax
import jax.numpy as jnp
from jax import lax
from jax.experimental import pallas as pl
from jax.experimental.pallas import tpu as pltpu
from jax.experimental.pallas import tpu_sc as plsc

N_NODES = 10000
N_EDGES = 320000
D = 128

NC = 2
NS = 16
NW = NC * NS
CHUNK = 80
GRP = 6
N_GRP = 21
CH_PER_W = GRP * N_GRP
E_PAD = NW * CH_PER_W * CHUNK
NP = 10112
ROWS_PER_SUB = NP // NS
PAD_ROW = N_NODES


def _mesh():
    return plsc.VectorSubcoreMesh(core_axis_name="c", subcore_axis_name="s")


@functools.lru_cache(maxsize=None)
def _sc_agg():
    @functools.partial(
        pl.kernel, mesh=_mesh(),
        out_type=jax.ShapeDtypeStruct((NC, NP, D), jnp.float32),
        scratch_types=[
            pltpu.VMEM((2, GRP, CHUNK), jnp.int32),
            pltpu.VMEM((2, GRP, CHUNK), jnp.int32),
            pltpu.VMEM((3, CHUNK, D), jnp.float32),
            pltpu.VMEM_SHARED((NP, D), jnp.float32),
            pltpu.SemaphoreType.DMA,
            pltpu.SemaphoreType.DMA,
            pltpu.SemaphoreType.DMA,
            pltpu.SemaphoreType.DMA,
        ])
    def sc_kernel(x_hbm, src_hbm, dst_hbm, zeros_agg, agg_out,
                  src_v, dst_v, rows_v, agg_sh, sem0, sem1, sem2, isem):
        gsem = [sem0, sem1, sem2]
        cid = lax.axis_index("c")
        sid = lax.axis_index("s")
        wid = cid * NS + sid
        base = sid * ROWS_PER_SUB
        sl = pl.ds(base, ROWS_PER_SUB)

        pltpu.sync_copy(zeros_agg, agg_sh.at[sl])

        pltpu.sync_copy(src_hbm.at[wid, 0], src_v.at[0])
        pltpu.sync_copy(dst_hbm.at[wid, 0], dst_v.at[0])
        pltpu.make_async_copy(src_hbm.at[wid, 1], src_v.at[1], isem).start()
        pltpu.make_async_copy(dst_hbm.at[wid, 1], dst_v.at[1], isem).start()
        pltpu.make_async_copy(
            x_hbm.at[src_v.at[0, 0]], rows_v.at[0], gsem[0]).start()
        pltpu.make_async_copy(
            x_hbm.at[src_v.at[0, 1]], rows_v.at[1], gsem[1]).start()
        plsc.subcore_barrier()

        @pl.loop(0, N_GRP)
        def _(g):
            p = g % 2
            for j in range(GRP):
                b = j % 3
                nb = (j + 2) % 3
                pltpu.make_async_copy(
                    x_hbm.at[src_v.at[p, j]], rows_v.at[b], gsem[b]).wait()
                if j + 2 < GRP:
                    pltpu.make_async_copy(
                        x_hbm.at[src_v.at[p, j + 2]], rows_v.at[nb],
                        gsem[nb]).start()
                elif j + 2 == GRP:
                    @pl.when(g + 1 < N_GRP)
                    def _():
                        pltpu.make_async_copy(
                            src_hbm.at[wid, g + 1], src_v.at[1 - p],
                            isem).wait()
                        pltpu.make_async_copy(
                            dst_hbm.at[wid, g + 1], dst_v.at[1 - p],
                            isem).wait()
                        pltpu.make_async_copy(
                            x_hbm.at[src_v.at[1 - p, 0]], rows_v.at[nb],
                            gsem[nb]).start()
                else:
                    @pl.when(g + 1 < N_GRP)
                    def _():
                        pltpu.make_async_copy(
                            x_hbm.at[src_v.at[1 - p, 1]], rows_v.at[nb],
                            gsem[nb]).start()
                pltpu.sync_copy(rows_v.at[b], agg_sh.at[dst_v.at[p, j]],
                                add=True)

            @pl.when(g + 2 < N_GRP)
            def _():
                pltpu.make_async_copy(
                    src_hbm.at[wid, g + 2], src_v.at[p], isem).start()
                pltpu.make_async_copy(
                    dst_hbm.at[wid, g + 2], dst_v.at[p], isem).start()

        plsc.subcore_barrier()
        pltpu.sync_copy(agg_sh.at[sl], agg_out.at[cid, sl])

    return sc_kernel


@functools.lru_cache(maxsize=None)
def _sc_deg():
    cp = pltpu.CompilerParams()
    if "needs_layout_passes" in pltpu.CompilerParams.__dataclass_fields__:
        cp = dataclasses.replace(cp, needs_layout_passes=False)

    @functools.partial(
        pl.kernel, mesh=_mesh(), compiler_params=cp,
        out_type=jax.ShapeDtypeStruct((NW, 1, NP), jnp.float32),
        scratch_types=[
            pltpu.VMEM((CH_PER_W, CHUNK), jnp.int32),
            pltpu.VMEM((NP,), jnp.float32),
        ])
    def deg_kernel(dst_hbm, zeros_np, deg_out, dst_v, degp_v):
        cid = lax.axis_index("c")
        sid = lax.axis_index("s")
        wid = cid * NS + sid

        pltpu.sync_copy(dst_hbm.at[wid], dst_v)
        pltpu.sync_copy(zeros_np, degp_v)
        ones16 = jnp.ones((16,), jnp.float32)

        @pl.loop(0, CH_PER_W)
        def _(j):
            for c in range(CHUNK // 16):
                vec = dst_v[j, pl.ds(c * 16, 16)]
                plsc.addupdate_scatter(degp_v, [vec], ones16)

        pltpu.sync_copy(degp_v, deg_out.at[wid, 0])

    return deg_kernel


def _tc_body(agg_ref, deg_ref, x_ref, wl_ref, wr_ref, b_ref, o_ref):
    n = o_ref.shape[0]
    agg = agg_ref[0, :n] + agg_ref[1, :n]
    deg = jnp.sum(deg_ref[:, 0, :n], axis=0)[:, None]
    mean = agg / jnp.maximum(deg, 1.0)
    h = (jnp.dot(mean, wl_ref[...], preferred_element_type=jnp.float32)
         + jnp.dot(x_ref[:n], wr_ref[...], preferred_element_type=jnp.float32)
         + b_ref[...])
    o_ref[...] = jnp.maximum(h, 0.0)


def _tc_layer(agg, deg, x, W_l, W_r, b, n_out):
    return pl.pallas_call(
        _tc_body,
        out_shape=jax.ShapeDtypeStruct((n_out, D), jnp.float32),
    )(agg, deg, x, W_l, W_r, b.reshape(1, D))


def kernel(node_features, edge_index, W_l1, W_r1, b1, W_l2, W_r2, b2):
    ei = edge_index.astype(jnp.int32)
    pad_dst = PAD_ROW + jnp.arange(E_PAD - N_EDGES, dtype=jnp.int32) % (NP - PAD_ROW)
    src_flat = jnp.concatenate([ei[0], jnp.zeros((E_PAD - N_EDGES,), jnp.int32)])
    dst_flat = jnp.concatenate([ei[1], pad_dst])
    src_p = src_flat.reshape(NW, N_GRP, GRP, CHUNK)
    dst_p = dst_flat.reshape(NW, N_GRP, GRP, CHUNK)
    dst_p3 = dst_flat.reshape(NW, CH_PER_W, CHUNK)

    zeros_agg = jnp.zeros((ROWS_PER_SUB, D), jnp.float32)
    zeros_np = jnp.zeros((NP,), jnp.float32)

    deg = _sc_deg()(dst_p3, zeros_np)
    agg1 = _sc_agg()(node_features, src_p, dst_p, zeros_agg)
    x2 = _tc_layer(agg1, deg, node_features, W_l1, W_r1, b1, N_NODES)
    agg2 = _sc_agg()(x2, src_p, dst_p, zeros_agg)
    return _tc_layer(agg2, deg, x2, W_l2, W_r2, b2, N_NODES)


# speedup vs baseline: 1.8805x; 1.8805x over previous
import dataclasses
import functools

import jax
import jax.numpy as jnp
from jax import lax
from jax.experimental import pallas as pl
from jax.experimental.pallas import tpu as pltpu
from jax.experimental.pallas import tpu_sc as plsc

N_NODES = 10000
N_EDGES = 320000
D = 128

NC = 2
NS = 16
NW = NC * NS
CHUNK = 80
GRP = 6
N_GRP = 21
CH_PER_W = GRP * N_GRP
E_PAD = NW * CH_PER_W * CHUNK
NP = 10112
ROWS_PER_SUB = NP // NS
PAD_ROW = N_NODES


def _mesh():
    return plsc.VectorSubcoreMesh(core_axis_name="c", subcore_axis_name="s")


@functools.lru_cache(maxsize=None)
def _sc_agg():
    @functools.partial(
        pl.kernel, mesh=_mesh(),
        out_type=jax.ShapeDtypeStruct((NC, NP, D), jnp.float32),
        scratch_types=[
            pltpu.VMEM((2, GRP, CHUNK), jnp.int32),
            pltpu.VMEM((2, GRP, CHUNK), jnp.int32),
            pltpu.VMEM((3, CHUNK, D), jnp.float32),
            pltpu.VMEM_SHARED((NP, D), jnp.float32),
            pltpu.SemaphoreType.DMA,
            pltpu.SemaphoreType.DMA,
            pltpu.SemaphoreType.DMA,
            pltpu.SemaphoreType.DMA,
        ])
    def sc_kernel(x_hbm, src_hbm, dst_hbm, zeros_agg, agg_out,
                  src_v, dst_v, rows_v, agg_sh, sem0, sem1, sem2, isem):
        gsem = [sem0, sem1, sem2]
        cid = lax.axis_index("c")
        sid = lax.axis_index("s")
        wid = cid * NS + sid
        base = sid * ROWS_PER_SUB
        sl = pl.ds(base, ROWS_PER_SUB)

        pltpu.sync_copy(zeros_agg, agg_sh.at[sl])

        pltpu.sync_copy(src_hbm.at[wid, 0], src_v.at[0])
        pltpu.sync_copy(dst_hbm.at[wid, 0], dst_v.at[0])
        pltpu.make_async_copy(src_hbm.at[wid, 1], src_v.at[1], isem).start()
        pltpu.make_async_copy(dst_hbm.at[wid, 1], dst_v.at[1], isem).start()
        pltpu.make_async_copy(
            x_hbm.at[src_v.at[0, 0]], rows_v.at[0], gsem[0]).start()
        pltpu.make_async_copy(
            x_hbm.at[src_v.at[0, 1]], rows_v.at[1], gsem[1]).start()
        plsc.subcore_barrier()

        @pl.loop(0, N_GRP)
        def _(g):
            p = g % 2
            for j in range(GRP):
                b = j % 3
                nb = (j + 2) % 3
                pltpu.make_async_copy(
                    x_hbm.at[src_v.at[p, j]], rows_v.at[b], gsem[b]).wait()
                if j + 2 < GRP:
                    pltpu.make_async_copy(
                        x_hbm.at[src_v.at[p, j + 2]], rows_v.at[nb],
                        gsem[nb]).start()
                elif j + 2 == GRP:
                    @pl.when(g + 1 < N_GRP)
                    def _():
                        pltpu.make_async_copy(
                            src_hbm.at[wid, g + 1], src_v.at[1 - p],
                            isem).wait()
                        pltpu.make_async_copy(
                            dst_hbm.at[wid, g + 1], dst_v.at[1 - p],
                            isem).wait()
                        pltpu.make_async_copy(
                            x_hbm.at[src_v.at[1 - p, 0]], rows_v.at[nb],
                            gsem[nb]).start()
                else:
                    @pl.when(g + 1 < N_GRP)
                    def _():
                        pltpu.make_async_copy(
                            x_hbm.at[src_v.at[1 - p, 1]], rows_v.at[nb],
                            gsem[nb]).start()
                pltpu.sync_copy(rows_v.at[b], agg_sh.at[dst_v.at[p, j]],
                                add=True)

            @pl.when(g + 2 < N_GRP)
            def _():
                pltpu.make_async_copy(
                    src_hbm.at[wid, g + 2], src_v.at[p], isem).start()
                pltpu.make_async_copy(
                    dst_hbm.at[wid, g + 2], dst_v.at[p], isem).start()

        plsc.subcore_barrier()
        pltpu.sync_copy(agg_sh.at[sl], agg_out.at[cid, sl])

    return sc_kernel


@functools.lru_cache(maxsize=None)
def _sc_deg():
    cp = pltpu.CompilerParams()
    if "needs_layout_passes" in pltpu.CompilerParams.__dataclass_fields__:
        cp = dataclasses.replace(cp, needs_layout_passes=False)

    @functools.partial(
        pl.kernel, mesh=_mesh(), compiler_params=cp,
        out_type=jax.ShapeDtypeStruct((NW, 1, NP), jnp.float32),
        scratch_types=[
            pltpu.VMEM((CH_PER_W, CHUNK), jnp.int32),
            pltpu.VMEM((NP,), jnp.float32),
        ])
    def deg_kernel(dst_hbm, zeros_np, deg_out, dst_v, degp_v):
        cid = lax.axis_index("c")
        sid = lax.axis_index("s")
        wid = cid * NS + sid

        pltpu.sync_copy(dst_hbm.at[wid], dst_v)
        pltpu.sync_copy(zeros_np, degp_v)
        ones16 = jnp.ones((16,), jnp.float32)

        @pl.loop(0, CH_PER_W)
        def _(j):
            for c in range(CHUNK // 16):
                vec = dst_v[j, pl.ds(c * 16, 16)]
                plsc.addupdate_scatter(degp_v, [vec], ones16)

        pltpu.sync_copy(degp_v, deg_out.at[wid, 0])

    return deg_kernel


def _tc_body(agg_ref, deg_ref, x_ref, wl_ref, wr_ref, b_ref, o_ref):
    n = o_ref.shape[0]
    agg = agg_ref[0, :n] + agg_ref[1, :n]
    deg = jnp.sum(deg_ref[:, 0, :n], axis=0)[:, None]
    mean = agg / jnp.maximum(deg, 1.0)
    h = (jnp.dot(mean, wl_ref[...], preferred_element_type=jnp.float32)
         + jnp.dot(x_ref[:n], wr_ref[...], preferred_element_type=jnp.float32)
         + b_ref[...])
    o_ref[...] = jnp.maximum(h, 0.0)


def _tc_layer(agg, deg, x, W_l, W_r, b, n_out):
    return pl.pallas_call(
        _tc_body,
        out_shape=jax.ShapeDtypeStruct((n_out, D), jnp.float32),
    )(agg, deg, x, W_l, W_r, b.reshape(1, D))


def kernel(node_features, edge_index, W_l1, W_r1, b1, W_l2, W_r2, b2):
    ei = edge_index.astype(jnp.int32)
    pad_dst = PAD_ROW + jnp.arange(E_PAD - N_EDGES, dtype=jnp.int32) % (NP - PAD_ROW)
    src_flat = jnp.concatenate([ei[0], jnp.arange(E_PAD - N_EDGES, dtype=jnp.int32) * 997 % N_NODES])
    dst_flat = jnp.concatenate([ei[1], pad_dst])
    src_p = src_flat.reshape(NW, N_GRP, GRP, CHUNK)
    dst_p = dst_flat.reshape(NW, N_GRP, GRP, CHUNK)
    dst_p3 = dst_flat.reshape(NW, CH_PER_W, CHUNK)

    zeros_agg = jnp.zeros((ROWS_PER_SUB, D), jnp.float32)
    zeros_np = jnp.zeros((NP,), jnp.float32)

    deg = _sc_deg()(dst_p3, zeros_np)
    agg1 = _sc_agg()(node_features, src_p, dst_p, zeros_agg)
    x2 = _tc_layer(agg1, deg, node_features, W_l1, W_r1, b1, N_NODES)
    agg2 = _sc_agg()(x2, src_p, dst_p, zeros_agg)
    return _tc_layer(agg2, deg, x2, W_l2, W_r2, b2, N_NODES)
